# baseline (speedup 1.0000x reference)
import functools

import numpy as np

import jax
import jax.numpy as jnp
from jax import lax
from jax.experimental import pallas as pl
from jax.experimental.pallas import tpu as pltpu
from jax.experimental.pallas import tpu_sc as plsc

N_NODES = 10000
N_EDGES = 160000
N_GRAPHS = 100
D = 32
H1 = 128
OUT_DIM = 4
ROUNDS = 5
EPS = 1e-10
SLOPE = 0.01

NC = 2
NS = 16
NW = NC * NS
EPW = N_EDGES // NW
CH = 1000
NCH = EPW // CH
RPW = 1000
NTI = N_NODES // RPW

PK = 4
ER = N_EDGES // PK
H4 = PK * H1
BE = 4000
BR = BE // PK
NEB = N_EDGES // BE

_f32 = functools.partial(jnp.asarray, dtype=jnp.float32)

_O1 = np.kron(np.eye(PK), np.ones((H1, 1))).astype(np.float32)
_E1 = np.kron(np.eye(PK), np.ones((1, H1))).astype(np.float32)
_O2 = np.kron(np.eye(PK), np.ones((D, 1))).astype(np.float32)
_E2 = np.kron(np.eye(PK), np.ones((1, D))).astype(np.float32)


@functools.cache
def _sc_kernels():
    mesh = plsc.VectorSubcoreMesh(core_axis_name="c", subcore_axis_name="s",
                                  num_cores=NC, num_subcores=NS)

    @functools.partial(
        pl.kernel,
        out_type=(jax.ShapeDtypeStruct((N_EDGES, D), jnp.float32),
                  jax.ShapeDtypeStruct((N_EDGES, D), jnp.float32)),
        mesh=mesh,
        compiler_params=pltpu.CompilerParams(use_tc_tiling_on_sc=False),
        scratch_types=[pltpu.VMEM((CH,), jnp.int32),
                       pltpu.VMEM((CH, D), jnp.float32),
                       pltpu.VMEM((CH, D), jnp.float32),
                       pltpu.SemaphoreType.DMA],
    )
    def sc_gather(p, idx_hbm, g0, g1, idx_v, r0, r1, sem):
        c = lax.axis_index("c")
        s = lax.axis_index("s")
        base = (s * jnp.int32(NC) + c) * jnp.int32(EPW)

        def body(i, carry):
            off = base + i * jnp.int32(CH)
            pltpu.sync_copy(idx_hbm.at[pl.ds(off, CH)], idx_v)
            cp0 = pltpu.async_copy(p.at[jnp.int32(0)].at[idx_v], r0, sem)
            cp1 = pltpu.async_copy(p.at[jnp.int32(1)].at[idx_v], r1, sem)
            cp0.wait()
            cp1.wait()
            pltpu.sync_copy(r0, g0.at[pl.ds(off, CH)])
            pltpu.sync_copy(r1, g1.at[pl.ds(off, CH)])
            return carry

        lax.fori_loop(jnp.int32(0), jnp.int32(NCH), body, jnp.int32(0))

    @functools.partial(
        pl.kernel,
        out_type=jax.ShapeDtypeStruct((NC, N_NODES, D), jnp.float32),
        mesh=mesh,
        compiler_params=pltpu.CompilerParams(use_tc_tiling_on_sc=False),
        scratch_types=[pltpu.VMEM((CH,), jnp.int32),
                       pltpu.VMEM((CH, D), jnp.float32),
                       pltpu.VMEM_SHARED((N_NODES, D), jnp.float32)],
    )
    def sc_scatter(pp, msg, idx_hbm, q, idx_v, rows_v, acc):
        c = lax.axis_index("c")
        s = lax.axis_index("s")
        rs = s * jnp.int32(RPW)

        @pl.when(s < NTI)
        def _():
            pltpu.sync_copy(pp.at[c].at[pl.ds(rs, RPW)], rows_v)
            pltpu.sync_copy(rows_v, acc.at[pl.ds(rs, RPW)])

        plsc.subcore_barrier()

        base = (s * jnp.int32(NC) + c) * jnp.int32(EPW)

        def body(i, carry):
            off = base + i * jnp.int32(CH)
            pltpu.sync_copy(idx_hbm.at[pl.ds(off, CH)], idx_v)
            pltpu.sync_copy(msg.at[pl.ds(off, CH)], rows_v)
            pltpu.sync_copy(rows_v, acc.at[idx_v], add=True)
            return carry

        lax.fori_loop(jnp.int32(0), jnp.int32(NCH), body, jnp.int32(0))
        plsc.subcore_barrier()

        @pl.when(s < NTI)
        def _():
            pltpu.sync_copy(acc.at[pl.ds(rs, RPW)], rows_v)
            pltpu.sync_copy(rows_v, q.at[c].at[pl.ds(rs, RPW)])

    return sc_gather, sc_scatter


PREC = lax.Precision.HIGHEST


def _dot(a, b):
    return jnp.dot(a, b, preferred_element_type=jnp.float32, precision=PREC)


def _leaky(x):
    return jnp.where(x >= 0, x, SLOPE * x)


def _group_ln(x, osum, ebc, inv_n, g, b):
    m = _dot(_dot(x, osum) * inv_n, ebc)
    d = x - m
    v = _dot(_dot(d * d, osum) * inv_n, ebc)
    return d * lax.rsqrt(v + 1e-5) * g + b


def _mlp_body(first, *refs):
    if first:
        (ec4, e1, w1b, b1, g1, bb1, o1, w2, b2, g2, bb2, o2, e2, out) = refs
        h = _dot(ec4[...], e1[...]) * w1b[...] + b1[...]
    else:
        (ga, gb, ec4, w1a, e1, w1b, b1, g1, bb1, o1,
         w2, b2, g2, bb2, o2, e2, out) = refs
        x = ga[...] + gb[...]
        h = (_dot(x, w1a[...])
             + _dot(ec4[...], e1[...]) * w1b[...] + b1[...])
    h = _leaky(_group_ln(h, o1[...], e1[...], 1.0 / H1, g1[...], bb1[...]))
    u = _dot(h, w2[...]) + b2[...]
    out[...] = _leaky(_group_ln(u, o2[...], e2[...], 1.0 / D,
                                g2[...], bb2[...]))


def _make_mlp(first):
    full = lambda i: (jnp.int32(0), jnp.int32(0))
    edge = lambda i: (i, jnp.int32(0))
    in_specs = []
    if not first:
        in_specs += [pl.BlockSpec((BR, H1), edge)] * 2
    in_specs += [pl.BlockSpec((BR, PK), edge)]
    if not first:
        in_specs += [pl.BlockSpec((H1, H4), full)]
    in_specs += [pl.BlockSpec((PK, H4), full)]
    in_specs += [pl.BlockSpec((1, H4), full)] * 4
    in_specs += [pl.BlockSpec((H4, PK), full)]
    in_specs += [pl.BlockSpec((H4, H1), full)]
    in_specs += [pl.BlockSpec((1, H1), full)] * 3
    in_specs += [pl.BlockSpec((H1, PK), full)]
    in_specs += [pl.BlockSpec((PK, H1), full)]
    return pl.pallas_call(
        functools.partial(_mlp_body, first),
        grid=(NEB,),
        in_specs=in_specs,
        out_specs=pl.BlockSpec((BR, H1), edge),
        out_shape=jax.ShapeDtypeStruct((ER, H1), jnp.float32),
    )


_mlp_first = _make_mlp(True)
_mlp = _make_mlp(False)


NB = 2000
NGB = N_NODES // NB


def _pool_body(ngi, p0, p1, ow1, ob1, ow2, ob2, out, acc):
    i = pl.program_id(0)

    @pl.when(i == 0)
    def _():
        acc[...] = jnp.zeros_like(acc)

    rows = p0[...] + p1[...]
    gids = ngi[0]
    giota = lax.broadcasted_iota(jnp.int32, (N_GRAPHS, NB), 0)
    oh = (giota == gids).astype(jnp.float32)
    acc[...] += _dot(oh, rows)

    @pl.when(i == NGB - 1)
    def _():
        ev = _dot(acc[...], ow1[...]) + ob1[...]
        ev = _dot(ev, ow2[...]) + ob2[...]
        sp = jnp.maximum(ev, 0.0) + jnp.log1p(jnp.exp(-jnp.abs(ev)))
        col = lax.broadcasted_iota(jnp.int32, (N_GRAPHS, OUT_DIM), 1)
        out[...] = jnp.where(col == 0, ev,
                             sp + EPS + (col == 2).astype(jnp.float32))


_pool_head = pl.pallas_call(
    _pool_body,
    grid=(NGB,),
    in_specs=[pl.BlockSpec((1, 1, NB), lambda i: (i, jnp.int32(0), jnp.int32(0))),
              pl.BlockSpec((NB, D), lambda i: (i, jnp.int32(0))),
              pl.BlockSpec((NB, D), lambda i: (i, jnp.int32(0))),
              pl.BlockSpec((D, H1), lambda i: (jnp.int32(0), jnp.int32(0))),
              pl.BlockSpec((1, H1), lambda i: (jnp.int32(0), jnp.int32(0))),
              pl.BlockSpec((H1, OUT_DIM), lambda i: (jnp.int32(0), jnp.int32(0))),
              pl.BlockSpec((1, OUT_DIM), lambda i: (jnp.int32(0), jnp.int32(0)))],
    out_specs=pl.BlockSpec((N_GRAPHS, OUT_DIM),
                           lambda i: (jnp.int32(0), jnp.int32(0))),
    out_shape=jax.ShapeDtypeStruct((N_GRAPHS, OUT_DIM), jnp.float32),
    scratch_shapes=[pltpu.VMEM((N_GRAPHS, D), jnp.float32)],
)


def _bdiag(w, n):
    a, b = w.shape
    out = jnp.zeros((n * a, n * b), w.dtype)
    for g in range(n):
        out = out.at[g * a:(g + 1) * a, g * b:(g + 1) * b].set(w)
    return out


def kernel(edge_coulomb, edge_lengths, node_from, node_to, node_graph_index,
           w1, b1, ln1_g, ln1_b, w2, b2, ln2_g, ln2_b, ow1, ob1, ow2, ob2):
    ec4 = _f32(edge_coulomb.reshape(N_EDGES)).reshape(ER, PK)
    nf = node_from.astype(jnp.int32)
    nt = node_to.astype(jnp.int32)
    ngi = node_graph_index.astype(jnp.int32).reshape(NGB, 1, NB)

    w1a4 = _bdiag(_f32(w1[:D]), PK)
    w1b4 = jnp.tile(_f32(w1[D:]), (1, PK))
    b1f = jnp.tile(_f32(b1).reshape(1, H1), (1, PK))
    g1f = jnp.tile(_f32(ln1_g).reshape(1, H1), (1, PK))
    bb1f = jnp.tile(_f32(ln1_b).reshape(1, H1), (1, PK))
    w2f4 = _bdiag(_f32(w2), PK)
    b2f = jnp.tile(_f32(b2).reshape(1, D), (1, PK))
    g2f = jnp.tile(_f32(ln2_g).reshape(1, D), (1, PK))
    bb2f = jnp.tile(_f32(ln2_b).reshape(1, D), (1, PK))
    ow1f = _f32(ow1)
    ob1f = _f32(ob1).reshape(1, H1)
    ow2f = _f32(ow2)
    ob2f = _f32(ob2).reshape(1, OUT_DIM)
    o1 = jnp.asarray(_O1)
    e1 = jnp.asarray(_E1)
    o2 = jnp.asarray(_O2)
    e2 = jnp.asarray(_E2)

    zeros = jnp.zeros((NC, N_NODES, D), jnp.float32)
    _sc_gather, _sc_scatter = _sc_kernels()

    msg = _mlp_first(ec4, e1, w1b4, b1f, g1f, bb1f, o1,
                     w2f4, b2f, g2f, bb2f, o2, e2)
    p = _sc_scatter(zeros, msg.reshape(N_EDGES, D), nt)
    for _ in range(ROUNDS - 1):
        g0, g1 = _sc_gather(p, nf)
        msg = _mlp(g0.reshape(ER, H1), g1.reshape(ER, H1), ec4,
                   w1a4, e1, w1b4, b1f, g1f, bb1f, o1,
                   w2f4, b2f, g2f, bb2f, o2, e2)
        p = _sc_scatter(p, msg.reshape(N_EDGES, D), nt)

    out = _pool_head(ngi, p[0], p[1], ow1f, ob1f, ow2f, ob2f)
    return out.astype(jnp.float64)


# speedup vs baseline: 2.0720x; 2.0720x over previous
import functools

import numpy as np

import jax
import jax.numpy as jnp
from jax import lax
from jax.experimental import pallas as pl
from jax.experimental.pallas import tpu as pltpu
from jax.experimental.pallas import tpu_sc as plsc

N_NODES = 10000
N_EDGES = 160000
N_GRAPHS = 100
D = 32
H1 = 128
OUT_DIM = 4
ROUNDS = 5
EPS = 1e-10
SLOPE = 0.01

NC = 2
NS = 16
NW = NC * NS
EPW = N_EDGES // NW
CH = 1000
NCH = EPW // CH
RPW = 1000
NTI = N_NODES // RPW

PK = 4
ER = N_EDGES // PK
H4 = PK * H1
BE = 4000
BR = BE // PK
NEB = N_EDGES // BE

_f32 = functools.partial(jnp.asarray, dtype=jnp.float32)

_O1 = np.kron(np.eye(PK), np.ones((H1, 1))).astype(np.float32)
_E1 = np.kron(np.eye(PK), np.ones((1, H1))).astype(np.float32)
_O2 = np.kron(np.eye(PK), np.ones((D, 1))).astype(np.float32)
_E2 = np.kron(np.eye(PK), np.ones((1, D))).astype(np.float32)


@functools.cache
def _sc_kernels():
    mesh = plsc.VectorSubcoreMesh(core_axis_name="c", subcore_axis_name="s",
                                  num_cores=NC, num_subcores=NS)

    @functools.partial(
        pl.kernel,
        out_type=(jax.ShapeDtypeStruct((N_EDGES, D), jnp.float32),
                  jax.ShapeDtypeStruct((N_EDGES, D), jnp.float32)),
        mesh=mesh,
        compiler_params=pltpu.CompilerParams(use_tc_tiling_on_sc=False),
        scratch_types=[pltpu.VMEM((CH,), jnp.int32),
                       pltpu.VMEM((CH, D), jnp.float32),
                       pltpu.VMEM((CH, D), jnp.float32),
                       pltpu.SemaphoreType.DMA],
    )
    def sc_gather(p, idx_hbm, g0, g1, idx_v, r0, r1, sem):
        c = lax.axis_index("c")
        s = lax.axis_index("s")
        base = (s * jnp.int32(NC) + c) * jnp.int32(EPW)

        def body(i, carry):
            off = base + i * jnp.int32(CH)
            pltpu.sync_copy(idx_hbm.at[pl.ds(off, CH)], idx_v)
            cp0 = pltpu.async_copy(p.at[jnp.int32(0)].at[idx_v], r0, sem)
            cp1 = pltpu.async_copy(p.at[jnp.int32(1)].at[idx_v], r1, sem)
            cp0.wait()
            cp1.wait()
            pltpu.sync_copy(r0, g0.at[pl.ds(off, CH)])
            pltpu.sync_copy(r1, g1.at[pl.ds(off, CH)])
            return carry

        lax.fori_loop(jnp.int32(0), jnp.int32(NCH), body, jnp.int32(0))

    @functools.partial(
        pl.kernel,
        out_type=jax.ShapeDtypeStruct((NC, N_NODES, D), jnp.float32),
        mesh=mesh,
        compiler_params=pltpu.CompilerParams(use_tc_tiling_on_sc=False),
        scratch_types=[pltpu.VMEM((CH,), jnp.int32),
                       pltpu.VMEM((CH, D), jnp.float32),
                       pltpu.VMEM_SHARED((N_NODES, D), jnp.float32)],
    )
    def sc_scatter(pp, msg, idx_hbm, q, idx_v, rows_v, acc):
        c = lax.axis_index("c")
        s = lax.axis_index("s")
        rs = s * jnp.int32(RPW)

        @pl.when(s < NTI)
        def _():
            pltpu.sync_copy(pp.at[c].at[pl.ds(rs, RPW)], rows_v)
            pltpu.sync_copy(rows_v, acc.at[pl.ds(rs, RPW)])

        plsc.subcore_barrier()

        base = (s * jnp.int32(NC) + c) * jnp.int32(EPW)

        def body(i, carry):
            off = base + i * jnp.int32(CH)
            pltpu.sync_copy(idx_hbm.at[pl.ds(off, CH)], idx_v)
            pltpu.sync_copy(msg.at[pl.ds(off, CH)], rows_v)
            pltpu.sync_copy(rows_v, acc.at[idx_v], add=True)
            return carry

        lax.fori_loop(jnp.int32(0), jnp.int32(NCH), body, jnp.int32(0))
        plsc.subcore_barrier()

        @pl.when(s < NTI)
        def _():
            pltpu.sync_copy(acc.at[pl.ds(rs, RPW)], rows_v)
            pltpu.sync_copy(rows_v, q.at[c].at[pl.ds(rs, RPW)])

    return sc_gather, sc_scatter


def _dot(a, b, prec=lax.Precision.HIGHEST):
    return jnp.dot(a, b, preferred_element_type=jnp.float32, precision=prec)


def _leaky(x):
    return jnp.where(x >= 0, x, SLOPE * x)


def _group_ln(x, n, g, b):
    parts = []
    for k in range(PK):
        xg = x[:, n * k:n * (k + 1)]
        m = jnp.mean(xg, axis=-1, keepdims=True)
        d = xg - m
        v = jnp.mean(d * d, axis=-1, keepdims=True)
        parts.append(d * lax.rsqrt(v + 1e-5))
    return jnp.concatenate(parts, axis=1) * g + b


def _mlp_body(first, *refs):
    if first:
        (ec4, w1b, b1, g1, bb1, w2, b2, g2, bb2, out) = refs
        h = b1[...]
    else:
        (ga, gb, ec4, w1a, w1b, b1, g1, bb1,
         w2, b2, g2, bb2, out) = refs
        x = ga[...] + gb[...]
        h = _dot(x, w1a[...]) + b1[...]
    ecv = ec4[...]
    w1bv = w1b[...]
    ecw = jnp.concatenate(
        [ecv[:, k:k + 1] * w1bv for k in range(PK)], axis=1)
    h = h + ecw
    h = _leaky(_group_ln(h, H1, g1[...], bb1[...]))
    u = _dot(h, w2[...]) + b2[...]
    out[...] = _leaky(_group_ln(u, D, g2[...], bb2[...]))


def _make_mlp(first):
    full = lambda i: (jnp.int32(0), jnp.int32(0))
    edge = lambda i: (i, jnp.int32(0))
    in_specs = []
    if not first:
        in_specs += [pl.BlockSpec((BR, H1), edge)] * 2
    in_specs += [pl.BlockSpec((BR, PK), edge)]
    if not first:
        in_specs += [pl.BlockSpec((H1, H4), full)]
    in_specs += [pl.BlockSpec((1, H1), full)]
    in_specs += [pl.BlockSpec((1, H4), full)] * 3
    in_specs += [pl.BlockSpec((H4, H1), full)]
    in_specs += [pl.BlockSpec((1, H1), full)] * 3
    return pl.pallas_call(
        functools.partial(_mlp_body, first),
        grid=(NEB,),
        in_specs=in_specs,
        out_specs=pl.BlockSpec((BR, H1), edge),
        out_shape=jax.ShapeDtypeStruct((ER, H1), jnp.float32),
    )


_mlp_first = _make_mlp(True)
_mlp = _make_mlp(False)


NB = 2000
NGB = N_NODES // NB


def _pool_body(ngi, p0, p1, ow1, ob1, ow2, ob2, out, acc):
    i = pl.program_id(0)

    @pl.when(i == 0)
    def _():
        acc[...] = jnp.zeros_like(acc)

    rows = p0[...] + p1[...]
    gids = ngi[0]
    giota = lax.broadcasted_iota(jnp.int32, (N_GRAPHS, NB), 0)
    oh = (giota == gids).astype(jnp.float32)
    acc[...] += _dot(oh, rows, lax.Precision.HIGHEST)

    @pl.when(i == NGB - 1)
    def _():
        ev = _dot(acc[...], ow1[...], lax.Precision.HIGHEST) + ob1[...]
        ev = _dot(ev, ow2[...], lax.Precision.HIGHEST) + ob2[...]
        sp = jnp.maximum(ev, 0.0) + jnp.log1p(jnp.exp(-jnp.abs(ev)))
        col = lax.broadcasted_iota(jnp.int32, (N_GRAPHS, OUT_DIM), 1)
        out[...] = jnp.where(col == 0, ev,
                             sp + EPS + (col == 2).astype(jnp.float32))


_pool_head = pl.pallas_call(
    _pool_body,
    grid=(NGB,),
    in_specs=[pl.BlockSpec((1, 1, NB), lambda i: (i, jnp.int32(0), jnp.int32(0))),
              pl.BlockSpec((NB, D), lambda i: (i, jnp.int32(0))),
              pl.BlockSpec((NB, D), lambda i: (i, jnp.int32(0))),
              pl.BlockSpec((D, H1), lambda i: (jnp.int32(0), jnp.int32(0))),
              pl.BlockSpec((1, H1), lambda i: (jnp.int32(0), jnp.int32(0))),
              pl.BlockSpec((H1, OUT_DIM), lambda i: (jnp.int32(0), jnp.int32(0))),
              pl.BlockSpec((1, OUT_DIM), lambda i: (jnp.int32(0), jnp.int32(0)))],
    out_specs=pl.BlockSpec((N_GRAPHS, OUT_DIM),
                           lambda i: (jnp.int32(0), jnp.int32(0))),
    out_shape=jax.ShapeDtypeStruct((N_GRAPHS, OUT_DIM), jnp.float32),
    scratch_shapes=[pltpu.VMEM((N_GRAPHS, D), jnp.float32)],
)


def _bdiag(w, n):
    a, b = w.shape
    out = jnp.zeros((n * a, n * b), w.dtype)
    for g in range(n):
        out = out.at[g * a:(g + 1) * a, g * b:(g + 1) * b].set(w)
    return out


def kernel(edge_coulomb, edge_lengths, node_from, node_to, node_graph_index,
           w1, b1, ln1_g, ln1_b, w2, b2, ln2_g, ln2_b, ow1, ob1, ow2, ob2):
    ec4 = _f32(edge_coulomb.reshape(N_EDGES)).reshape(ER, PK)
    nf = node_from.astype(jnp.int32)
    nt = node_to.astype(jnp.int32)
    ngi = node_graph_index.astype(jnp.int32).reshape(NGB, 1, NB)

    w1a4 = _bdiag(_f32(w1[:D]), PK)
    w1bf = _f32(w1[D:])
    b1f = jnp.tile(_f32(b1).reshape(1, H1), (1, PK))
    g1f = jnp.tile(_f32(ln1_g).reshape(1, H1), (1, PK))
    bb1f = jnp.tile(_f32(ln1_b).reshape(1, H1), (1, PK))
    w2f4 = _bdiag(_f32(w2), PK)
    b2f = jnp.tile(_f32(b2).reshape(1, D), (1, PK))
    g2f = jnp.tile(_f32(ln2_g).reshape(1, D), (1, PK))
    bb2f = jnp.tile(_f32(ln2_b).reshape(1, D), (1, PK))
    ow1f = _f32(ow1)
    ob1f = _f32(ob1).reshape(1, H1)
    ow2f = _f32(ow2)
    ob2f = _f32(ob2).reshape(1, OUT_DIM)
    zeros = jnp.zeros((NC, N_NODES, D), jnp.float32)
    _sc_gather, _sc_scatter = _sc_kernels()

    msg = _mlp_first(ec4, w1bf, b1f, g1f, bb1f,
                     w2f4, b2f, g2f, bb2f)
    p = _sc_scatter(zeros, msg.reshape(N_EDGES, D), nt)
    for _ in range(ROUNDS - 1):
        g0, g1 = _sc_gather(p, nf)
        msg = _mlp(g0.reshape(ER, H1), g1.reshape(ER, H1), ec4,
                   w1a4, w1bf, b1f, g1f, bb1f,
                   w2f4, b2f, g2f, bb2f)
        p = _sc_scatter(p, msg.reshape(N_EDGES, D), nt)

    out = _pool_head(ngi, p[0], p[1], ow1f, ob1f, ow2f, ob2f)
    return out.astype(jnp.float64)


# speedup vs baseline: 3.3846x; 1.6335x over previous
import functools

import numpy as np

import jax
import jax.numpy as jnp
from jax import lax
from jax.experimental import pallas as pl
from jax.experimental.pallas import tpu as pltpu
from jax.experimental.pallas import tpu_sc as plsc

N_NODES = 10000
N_EDGES = 160000
N_GRAPHS = 100
D = 32
H1 = 128
OUT_DIM = 4
ROUNDS = 5
EPS = 1e-10
SLOPE = 0.01

NC = 2
NS = 16
NW = NC * NS
EPW = N_EDGES // NW
CH = 1000
NCH = EPW // CH
RPW = 1000
NTI = N_NODES // RPW

PK = 4
ER = N_EDGES // PK
H4 = PK * H1
BE = 4000
BR = BE // PK
NEB = N_EDGES // BE

_f32 = functools.partial(jnp.asarray, dtype=jnp.float32)

_O1 = np.kron(np.eye(PK), np.ones((H1, 1))).astype(np.float32)
_E1 = np.kron(np.eye(PK), np.ones((1, H1))).astype(np.float32)
_O2 = np.kron(np.eye(PK), np.ones((D, 1))).astype(np.float32)
_E2 = np.kron(np.eye(PK), np.ones((1, D))).astype(np.float32)


@functools.cache
def _sc_kernels():
    mesh = plsc.VectorSubcoreMesh(core_axis_name="c", subcore_axis_name="s",
                                  num_cores=NC, num_subcores=NS)

    @functools.partial(
        pl.kernel,
        out_type=(jax.ShapeDtypeStruct((N_EDGES, D), jnp.float32),
                  jax.ShapeDtypeStruct((N_EDGES, D), jnp.float32)),
        mesh=mesh,
        compiler_params=pltpu.CompilerParams(use_tc_tiling_on_sc=False),
        scratch_types=[pltpu.VMEM((CH,), jnp.int32),
                       pltpu.VMEM((CH, D), jnp.float32),
                       pltpu.VMEM((CH, D), jnp.float32),
                       pltpu.SemaphoreType.DMA],
    )
    def sc_gather(p, idx_hbm, g0, g1, idx_v, r0, r1, sem):
        c = lax.axis_index("c")
        s = lax.axis_index("s")
        base = (s * jnp.int32(NC) + c) * jnp.int32(EPW)

        def body(i, carry):
            off = base + i * jnp.int32(CH)
            pltpu.sync_copy(idx_hbm.at[pl.ds(off, CH)], idx_v)
            cp0 = pltpu.async_copy(p.at[jnp.int32(0)].at[idx_v], r0, sem)
            cp1 = pltpu.async_copy(p.at[jnp.int32(1)].at[idx_v], r1, sem)
            cp0.wait()
            cp1.wait()
            pltpu.sync_copy(r0, g0.at[pl.ds(off, CH)])
            pltpu.sync_copy(r1, g1.at[pl.ds(off, CH)])
            return carry

        lax.fori_loop(jnp.int32(0), jnp.int32(NCH), body, jnp.int32(0))

    @functools.partial(
        pl.kernel,
        out_type=jax.ShapeDtypeStruct((NC, N_NODES, D), jnp.float32),
        mesh=mesh,
        compiler_params=pltpu.CompilerParams(use_tc_tiling_on_sc=False),
        scratch_types=[pltpu.VMEM((CH,), jnp.int32),
                       pltpu.VMEM((CH, D), jnp.float32),
                       pltpu.VMEM_SHARED((N_NODES, D), jnp.float32)],
    )
    def sc_scatter(pp, msg, idx_hbm, q, idx_v, rows_v, acc):
        c = lax.axis_index("c")
        s = lax.axis_index("s")
        rs = s * jnp.int32(RPW)

        @pl.when(s < NTI)
        def _():
            pltpu.sync_copy(pp.at[c].at[pl.ds(rs, RPW)], rows_v)
            pltpu.sync_copy(rows_v, acc.at[pl.ds(rs, RPW)])

        plsc.subcore_barrier()

        base = (s * jnp.int32(NC) + c) * jnp.int32(EPW)

        def body(i, carry):
            off = base + i * jnp.int32(CH)
            pltpu.sync_copy(idx_hbm.at[pl.ds(off, CH)], idx_v)
            pltpu.sync_copy(msg.at[pl.ds(off, CH)], rows_v)
            pltpu.sync_copy(rows_v, acc.at[idx_v], add=True)
            return carry

        lax.fori_loop(jnp.int32(0), jnp.int32(NCH), body, jnp.int32(0))
        plsc.subcore_barrier()

        @pl.when(s < NTI)
        def _():
            pltpu.sync_copy(acc.at[pl.ds(rs, RPW)], rows_v)
            pltpu.sync_copy(rows_v, q.at[c].at[pl.ds(rs, RPW)])

    return sc_gather, sc_scatter


def _dot(a, b, prec=lax.Precision.HIGHEST):
    return jnp.dot(a, b, preferred_element_type=jnp.float32, precision=prec)


def _split_bf16(w):
    hi = w.astype(jnp.bfloat16)
    lo = (w - hi.astype(jnp.float32)).astype(jnp.bfloat16)
    return hi, lo


def _dot3(x, w_hi, w_lo):
    xh = x.astype(jnp.bfloat16)
    xl = (x - xh.astype(jnp.float32)).astype(jnp.bfloat16)
    acc = jnp.dot(xh, w_lo, preferred_element_type=jnp.float32)
    acc = acc + jnp.dot(xl, w_hi, preferred_element_type=jnp.float32)
    return acc + jnp.dot(xh, w_hi, preferred_element_type=jnp.float32)


def _leaky(x):
    return jnp.where(x >= 0, x, SLOPE * x)


def _group_ln(x, n, g, b):
    parts = []
    for k in range(PK):
        xg = x[:, n * k:n * (k + 1)]
        m = jnp.mean(xg, axis=-1, keepdims=True)
        d = xg - m
        v = jnp.mean(d * d, axis=-1, keepdims=True)
        parts.append(d * lax.rsqrt(v + 1e-5))
    return jnp.concatenate(parts, axis=1) * g + b


def _mlp_body(first, *refs):
    if first:
        (ec4, w1b, b1, g1, bb1, w2h, w2l, b2, g2, bb2, out) = refs
        h = b1[...]
    else:
        (ga, gb, ec4, w1ah, w1al, w1b, b1, g1, bb1,
         w2h, w2l, b2, g2, bb2, out) = refs
        x = ga[...] + gb[...]
        h = _dot3(x, w1ah[...], w1al[...]) + b1[...]
    ecv = ec4[...]
    w1bv = w1b[...]
    ecw = jnp.concatenate(
        [ecv[:, k:k + 1] * w1bv for k in range(PK)], axis=1)
    h = h + ecw
    h = _leaky(_group_ln(h, H1, g1[...], bb1[...]))
    u = _dot3(h, w2h[...], w2l[...]) + b2[...]
    out[...] = _leaky(_group_ln(u, D, g2[...], bb2[...]))


def _make_mlp(first):
    full = lambda i: (jnp.int32(0), jnp.int32(0))
    edge = lambda i: (i, jnp.int32(0))
    in_specs = []
    if not first:
        in_specs += [pl.BlockSpec((BR, H1), edge)] * 2
    in_specs += [pl.BlockSpec((BR, PK), edge)]
    if not first:
        in_specs += [pl.BlockSpec((H1, H4), full)] * 2
    in_specs += [pl.BlockSpec((1, H1), full)]
    in_specs += [pl.BlockSpec((1, H4), full)] * 3
    in_specs += [pl.BlockSpec((H4, H1), full)] * 2
    in_specs += [pl.BlockSpec((1, H1), full)] * 3
    return pl.pallas_call(
        functools.partial(_mlp_body, first),
        grid=(NEB,),
        in_specs=in_specs,
        out_specs=pl.BlockSpec((BR, H1), edge),
        out_shape=jax.ShapeDtypeStruct((ER, H1), jnp.float32),
    )


_mlp_first = _make_mlp(True)
_mlp = _make_mlp(False)


NB = 2000
NGB = N_NODES // NB


def _pool_body(ngi, p0, p1, ow1, ob1, ow2, ob2, out, acc):
    i = pl.program_id(0)

    @pl.when(i == 0)
    def _():
        acc[...] = jnp.zeros_like(acc)

    rows = p0[...] + p1[...]
    gids = ngi[0]
    giota = lax.broadcasted_iota(jnp.int32, (N_GRAPHS, NB), 0)
    oh = (giota == gids).astype(jnp.float32)
    acc[...] += _dot(oh, rows, lax.Precision.HIGHEST)

    @pl.when(i == NGB - 1)
    def _():
        ev = _dot(acc[...], ow1[...], lax.Precision.HIGHEST) + ob1[...]
        ev = _dot(ev, ow2[...], lax.Precision.HIGHEST) + ob2[...]
        sp = jnp.maximum(ev, 0.0) + jnp.log1p(jnp.exp(-jnp.abs(ev)))
        col = lax.broadcasted_iota(jnp.int32, (N_GRAPHS, OUT_DIM), 1)
        out[...] = jnp.where(col == 0, ev,
                             sp + EPS + (col == 2).astype(jnp.float32))


_pool_head = pl.pallas_call(
    _pool_body,
    grid=(NGB,),
    in_specs=[pl.BlockSpec((1, 1, NB), lambda i: (i, jnp.int32(0), jnp.int32(0))),
              pl.BlockSpec((NB, D), lambda i: (i, jnp.int32(0))),
              pl.BlockSpec((NB, D), lambda i: (i, jnp.int32(0))),
              pl.BlockSpec((D, H1), lambda i: (jnp.int32(0), jnp.int32(0))),
              pl.BlockSpec((1, H1), lambda i: (jnp.int32(0), jnp.int32(0))),
              pl.BlockSpec((H1, OUT_DIM), lambda i: (jnp.int32(0), jnp.int32(0))),
              pl.BlockSpec((1, OUT_DIM), lambda i: (jnp.int32(0), jnp.int32(0)))],
    out_specs=pl.BlockSpec((N_GRAPHS, OUT_DIM),
                           lambda i: (jnp.int32(0), jnp.int32(0))),
    out_shape=jax.ShapeDtypeStruct((N_GRAPHS, OUT_DIM), jnp.float32),
    scratch_shapes=[pltpu.VMEM((N_GRAPHS, D), jnp.float32)],
)


def _bdiag(w, n):
    a, b = w.shape
    out = jnp.zeros((n * a, n * b), w.dtype)
    for g in range(n):
        out = out.at[g * a:(g + 1) * a, g * b:(g + 1) * b].set(w)
    return out


def kernel(edge_coulomb, edge_lengths, node_from, node_to, node_graph_index,
           w1, b1, ln1_g, ln1_b, w2, b2, ln2_g, ln2_b, ow1, ob1, ow2, ob2):
    ec4 = _f32(edge_coulomb.reshape(N_EDGES)).reshape(ER, PK)
    nf = node_from.astype(jnp.int32)
    nt = node_to.astype(jnp.int32)
    ngi = node_graph_index.astype(jnp.int32).reshape(NGB, 1, NB)

    w1a4h, w1a4l = _split_bf16(_bdiag(_f32(w1[:D]), PK))
    w1bf = _f32(w1[D:])
    b1f = jnp.tile(_f32(b1).reshape(1, H1), (1, PK))
    g1f = jnp.tile(_f32(ln1_g).reshape(1, H1), (1, PK))
    bb1f = jnp.tile(_f32(ln1_b).reshape(1, H1), (1, PK))
    w2f4h, w2f4l = _split_bf16(_bdiag(_f32(w2), PK))
    b2f = jnp.tile(_f32(b2).reshape(1, D), (1, PK))
    g2f = jnp.tile(_f32(ln2_g).reshape(1, D), (1, PK))
    bb2f = jnp.tile(_f32(ln2_b).reshape(1, D), (1, PK))
    ow1f = _f32(ow1)
    ob1f = _f32(ob1).reshape(1, H1)
    ow2f = _f32(ow2)
    ob2f = _f32(ob2).reshape(1, OUT_DIM)
    zeros = jnp.zeros((NC, N_NODES, D), jnp.float32)
    _sc_gather, _sc_scatter = _sc_kernels()

    msg = _mlp_first(ec4, w1bf, b1f, g1f, bb1f,
                     w2f4h, w2f4l, b2f, g2f, bb2f)
    p = _sc_scatter(zeros, msg.reshape(N_EDGES, D), nt)
    for _ in range(ROUNDS - 1):
        g0, g1 = _sc_gather(p, nf)
        msg = _mlp(g0.reshape(ER, H1), g1.reshape(ER, H1), ec4,
                   w1a4h, w1a4l, w1bf, b1f, g1f, bb1f,
                   w2f4h, w2f4l, b2f, g2f, bb2f)
        p = _sc_scatter(p, msg.reshape(N_EDGES, D), nt)

    out = _pool_head(ngi, p[0], p[1], ow1f, ob1f, ow2f, ob2f)
    return out.astype(jnp.float64)
